# baseline (speedup 1.0000x reference)
import functools

import jax
import jax.numpy as jnp
from jax import lax
from jax.experimental import pallas as pl
from jax.experimental.pallas import tpu as pltpu
from jax.experimental.pallas import tpu_sc as plsc

_B = 4
_C = 384
_H = 224
_W = 224
_HW = _H * _W
_R = 48
_E = 64
_K = 8
_EPS = 1e-5

_TCB = 3
_HBLK = 14
_CPB = _H // _HBLK
_NCHUNK = _TCB * _CPB
_NBUF = 4

_NW = 32
_RPW = _H // _NW
_PIX = 112
_NCK = _RPW * 2
_NG = _C // 16


def _pool_tc_body(xt_ref, sums_out, r0, r1, r2, r3, sems):
    rings = (r0, r1, r2, r3)

    def start(i, j):
        b = i // _CPB
        h = i % _CPB
        pltpu.make_async_copy(
            xt_ref.at[b, pl.ds(h * _HBLK, _HBLK)],
            rings[j],
            sems.at[j],
        ).start()

    for i in range(_NBUF - 1):
        start(i, i)

    def group(g, _):
        for j in range(_NBUF):
            i = _NBUF * g + j
            pltpu.make_async_copy(
                xt_ref.at[0, pl.ds(0, _HBLK)],
                rings[j],
                sems.at[j],
            ).wait()

            @pl.when(i + _NBUF - 1 < _NCHUNK)
            def _prefetch():
                start(i + _NBUF - 1, (j + _NBUF - 1) % _NBUF)

            s = jnp.sum(rings[j][...], axis=0)
            part = jnp.sum(s, axis=0)
            b = i // _CPB

            @pl.when(i % _CPB == 0)
            def _init():
                sums_out[b, :] = part

            @pl.when(i % _CPB != 0)
            def _acc():
                sums_out[b, :] += part
        return _

    jax.lax.fori_loop(0, _NCHUNK // _NBUF, group, None)


def _pool_sc_body(xt_hbm, out_hbm, b0, b1, acc_v, sem0, sem1):
    cid = lax.axis_index("c")
    sid = lax.axis_index("s")
    wid = sid * 2 + cid
    base = wid * _RPW
    bufs = (b0, b1)
    sems = (sem0, sem1)

    def start(k, slot):
        r = base + k // 2
        h0 = (k % 2) * _PIX
        pltpu.make_async_copy(
            xt_hbm.at[_B - 1, r, pl.ds(h0, _PIX)],
            bufs[slot],
            sems[slot],
        ).start()

    def wait(slot):
        pltpu.make_async_copy(
            xt_hbm.at[_B - 1, 0, pl.ds(0, _PIX)],
            bufs[slot],
            sems[slot],
        ).wait()

    def consume(slot, accs):
        buf = bufs[slot]

        def px(p, a):
            return tuple(a[g] + buf[p, pl.ds(16 * g, 16)] for g in range(_NG))

        return jax.lax.fori_loop(0, _PIX, px, accs)

    start(0, 0)
    start(1, 1)
    accs = tuple(jnp.zeros((16,), jnp.float32) for _ in range(_NG))

    def grp(g, accs):
        k0 = 2 * g
        wait(0)

        @pl.when(k0 + 2 < _NCK)
        def _p0():
            start(k0 + 2, 0)

        accs = consume(0, accs)
        wait(1)

        @pl.when(k0 + 3 < _NCK)
        def _p1():
            start(k0 + 3, 1)

        accs = consume(1, accs)
        return accs

    accs = jax.lax.fori_loop(0, _NCK // 2, grp, accs)
    for g in range(_NG):
        acc_v[pl.ds(16 * g, 16)] = accs[g]
    pltpu.sync_copy(acc_v, out_hbm.at[wid])


def _route_body(tc_ref, sc_ref, w1_ref, b1_ref, w2_ref, b2_ref,
                vals_ref, idx_ref):
    b3 = jnp.sum(sc_ref[...], axis=0)[None, :]
    pooled = jnp.concatenate([tc_ref[...], b3], axis=0)
    hid = jax.lax.dot_general(pooled, w1_ref[...], (((1,), (1,)), ((), ())),
                              preferred_element_type=jnp.float32)
    hid = hid + b1_ref[...]
    hid = hid * jax.nn.sigmoid(hid)
    logits = jax.lax.dot_general(hid, w2_ref[...], (((1,), (1,)), ((), ())),
                                 preferred_element_type=jnp.float32)
    logits = logits + b2_ref[...]
    m = jnp.max(logits, axis=1, keepdims=True)
    e = jnp.exp(logits - m)
    probs = e / jnp.sum(e, axis=1, keepdims=True)

    pa = probs[:, :, None]
    pb = probs[:, None, :]
    ks = jax.lax.broadcasted_iota(jnp.int32, (_B, _E, _E), 1)
    js = jax.lax.broadcasted_iota(jnp.int32, (_B, _E, _E), 2)
    beats = (pa > pb) | ((pa == pb) & (ks < js))
    rank = jnp.sum(beats.astype(jnp.int32), axis=1)

    iota = jax.lax.broadcasted_iota(jnp.int32, (_B, _E), 1)
    vals = []
    idxs = []
    for s in range(_K):
        sel = rank == s
        vals.append(jnp.sum(jnp.where(sel, probs, 0.0), axis=1, keepdims=True))
        idxs.append(jnp.sum(jnp.where(sel, iota, 0), axis=1, keepdims=True))
    v = jnp.concatenate(vals, axis=1)
    i = jnp.concatenate(idxs, axis=1)
    ssum = jnp.sum(v, axis=1, keepdims=True) + 1e-6
    vals_ref[...] = v / ssum
    idx_ref[...] = i


@jax.jit
def kernel(x, W1, gamma1, beta1, mean1, var1, W2, gamma2, beta2, mean2, var2):
    s1 = gamma1 * jax.lax.rsqrt(var1 + _EPS)
    s2 = gamma2 * jax.lax.rsqrt(var2 + _EPS)
    w1 = (W1 * s1[:, None]) * (1.0 / _HW)
    b1 = (beta1 - mean1 * s1)[None, :]
    w2 = W2 * s2[:, None]
    b2 = (beta2 - mean2 * s2)[None, :]

    xt = jnp.transpose(x, (0, 2, 3, 1))

    tc_sums = pl.pallas_call(
        _pool_tc_body,
        in_specs=[pl.BlockSpec(memory_space=pl.ANY)],
        out_specs=pl.BlockSpec(memory_space=pltpu.VMEM),
        out_shape=jax.ShapeDtypeStruct((_TCB, _C), jnp.float32),
        scratch_shapes=[
            pltpu.VMEM((_HBLK, _W, _C), jnp.float32),
            pltpu.VMEM((_HBLK, _W, _C), jnp.float32),
            pltpu.VMEM((_HBLK, _W, _C), jnp.float32),
            pltpu.VMEM((_HBLK, _W, _C), jnp.float32),
            pltpu.SemaphoreType.DMA((_NBUF,)),
        ],
    )(xt)

    sc_pool = functools.partial(
        pl.kernel,
        out_type=jax.ShapeDtypeStruct((_NW, _C), jnp.float32),
        mesh=plsc.VectorSubcoreMesh(core_axis_name="c", subcore_axis_name="s"),
        scratch_types=[
            pltpu.VMEM((_PIX, _C), jnp.float32),
            pltpu.VMEM((_PIX, _C), jnp.float32),
            pltpu.VMEM((_C,), jnp.float32),
            pltpu.SemaphoreType.DMA,
            pltpu.SemaphoreType.DMA,
        ],
    )(_pool_sc_body)
    sc_parts = sc_pool(xt)

    vals, idxs = pl.pallas_call(
        _route_body,
        out_shape=(
            jax.ShapeDtypeStruct((_B, _K), jnp.float32),
            jax.ShapeDtypeStruct((_B, _K), jnp.int32),
        ),
    )(tc_sums, sc_parts, w1, b1, w2, b2)
    return vals, idxs


# speedup vs baseline: 1.1435x; 1.1435x over previous
import jax
import jax.numpy as jnp
from jax.experimental import pallas as pl
from jax.experimental.pallas import tpu as pltpu

_B = 4
_C = 384
_H = 224
_W = 224
_HW = _H * _W
_R = 48
_E = 64
_K = 8
_EPS = 1e-5

_HBLK = 7
_NCHUNK = _H // _HBLK
_NBUF = 4


def _route(pooled, w1_ref, b1_ref, w2_ref, b2_ref, vals_ref, idx_ref):
    hid = jax.lax.dot_general(pooled, w1_ref[...], (((1,), (1,)), ((), ())),
                              preferred_element_type=jnp.float32)
    hid = hid + b1_ref[...]
    hid = hid * jax.nn.sigmoid(hid)
    logits = jax.lax.dot_general(hid, w2_ref[...], (((1,), (1,)), ((), ())),
                                 preferred_element_type=jnp.float32)
    logits = logits + b2_ref[...]
    m = jnp.max(logits, axis=1, keepdims=True)
    e = jnp.exp(logits - m)
    probs = e / jnp.sum(e, axis=1, keepdims=True)

    pa = probs[:, :, None]
    pb = probs[:, None, :]
    ks = jax.lax.broadcasted_iota(jnp.int32, (_B, _E, _E), 1)
    js = jax.lax.broadcasted_iota(jnp.int32, (_B, _E, _E), 2)
    beats = (pa > pb) | ((pa == pb) & (ks < js))
    rank = jnp.sum(beats.astype(jnp.int32), axis=1)

    iota = jax.lax.broadcasted_iota(jnp.int32, (_B, _E), 1)
    vals = []
    idxs = []
    for s in range(_K):
        sel = rank == s
        vals.append(jnp.sum(jnp.where(sel, probs, 0.0), axis=1, keepdims=True))
        idxs.append(jnp.sum(jnp.where(sel, iota, 0), axis=1, keepdims=True))
    v = jnp.concatenate(vals, axis=1)
    i = jnp.concatenate(idxs, axis=1)
    ssum = jnp.sum(v, axis=1, keepdims=True) + 1e-6
    vals_ref[...] = v / ssum
    idx_ref[...] = i


def _body(xt_ref, w1_ref, b1_ref, w2_ref, b2_ref, vals_ref, idx_ref,
          r0, r1, r2, r3, sums_ref, sems):
    rings = (r0, r1, r2, r3)

    def start(i, j):
        pltpu.make_async_copy(
            xt_ref.at[:, pl.ds(i * _HBLK, _HBLK)],
            rings[j],
            sems.at[j],
        ).start()

    for i in range(_NBUF - 1):
        start(i, i)

    def group(g, _):
        for j in range(_NBUF):
            i = _NBUF * g + j
            pltpu.make_async_copy(
                xt_ref.at[:, pl.ds(0, _HBLK)],
                rings[j],
                sems.at[j],
            ).wait()

            @pl.when(i + _NBUF - 1 < _NCHUNK)
            def _prefetch():
                start(i + _NBUF - 1, (j + _NBUF - 1) % _NBUF)

            s = jnp.sum(rings[j][...], axis=1)
            part = jnp.sum(s, axis=1)

            @pl.when(i == 0)
            def _init():
                sums_ref[...] = part

            @pl.when(i != 0)
            def _acc():
                sums_ref[...] += part
        return _

    jax.lax.fori_loop(0, _NCHUNK // _NBUF, group, None)
    _route(sums_ref[...], w1_ref, b1_ref, w2_ref, b2_ref, vals_ref, idx_ref)


@jax.jit
def kernel(x, W1, gamma1, beta1, mean1, var1, W2, gamma2, beta2, mean2, var2):
    s1 = gamma1 * jax.lax.rsqrt(var1 + _EPS)
    s2 = gamma2 * jax.lax.rsqrt(var2 + _EPS)
    w1 = (W1 * s1[:, None]) * (1.0 / _HW)
    b1 = (beta1 - mean1 * s1)[None, :]
    w2 = W2 * s2[:, None]
    b2 = (beta2 - mean2 * s2)[None, :]

    xt = jnp.transpose(x, (0, 2, 3, 1))
    vals, idxs = pl.pallas_call(
        _body,
        in_specs=[
            pl.BlockSpec(memory_space=pl.ANY),
            pl.BlockSpec(memory_space=pltpu.VMEM),
            pl.BlockSpec(memory_space=pltpu.VMEM),
            pl.BlockSpec(memory_space=pltpu.VMEM),
            pl.BlockSpec(memory_space=pltpu.VMEM),
        ],
        out_specs=(
            pl.BlockSpec(memory_space=pltpu.VMEM),
            pl.BlockSpec(memory_space=pltpu.VMEM),
        ),
        out_shape=(
            jax.ShapeDtypeStruct((_B, _K), jnp.float32),
            jax.ShapeDtypeStruct((_B, _K), jnp.int32),
        ),
        scratch_shapes=[
            pltpu.VMEM((_B, _HBLK, _W, _C), jnp.float32),
            pltpu.VMEM((_B, _HBLK, _W, _C), jnp.float32),
            pltpu.VMEM((_B, _HBLK, _W, _C), jnp.float32),
            pltpu.VMEM((_B, _HBLK, _W, _C), jnp.float32),
            pltpu.VMEM((_B, _C), jnp.float32),
            pltpu.SemaphoreType.DMA((_NBUF,)),
        ],
    )(xt, w1, b1, w2, b2)
    return vals, idxs


# speedup vs baseline: 1.1493x; 1.0051x over previous
import jax
import jax.numpy as jnp
from jax.experimental import pallas as pl
from jax.experimental.pallas import tpu as pltpu

_B = 4
_C = 384
_H = 224
_W = 224
_HW = _H * _W
_R = 48
_E = 64
_K = 8
_EPS = 1e-5

_HBLK = 14
_NH = _H // _HBLK
_NBUF = 4


def _route(pooled, w1_ref, b1_ref, w2_ref, b2_ref, vals_ref, idx_ref):
    hid = jax.lax.dot_general(pooled, w1_ref[...], (((1,), (1,)), ((), ())),
                              preferred_element_type=jnp.float32)
    hid = hid + b1_ref[...]
    hid = hid * jax.nn.sigmoid(hid)
    logits = jax.lax.dot_general(hid, w2_ref[...], (((1,), (1,)), ((), ())),
                                 preferred_element_type=jnp.float32)
    logits = logits + b2_ref[...]
    m = jnp.max(logits, axis=1, keepdims=True)
    e = jnp.exp(logits - m)
    probs = e / jnp.sum(e, axis=1, keepdims=True)

    pa = probs[:, :, None]
    pb = probs[:, None, :]
    ks = jax.lax.broadcasted_iota(jnp.int32, (_B, _E, _E), 1)
    js = jax.lax.broadcasted_iota(jnp.int32, (_B, _E, _E), 2)
    beats = (pa > pb) | ((pa == pb) & (ks < js))
    rank = jnp.sum(beats.astype(jnp.int32), axis=1)

    iota = jax.lax.broadcasted_iota(jnp.int32, (_B, _E), 1)
    vals = []
    idxs = []
    for s in range(_K):
        sel = rank == s
        vals.append(jnp.sum(jnp.where(sel, probs, 0.0), axis=1, keepdims=True))
        idxs.append(jnp.sum(jnp.where(sel, iota, 0), axis=1, keepdims=True))
    v = jnp.concatenate(vals, axis=1)
    i = jnp.concatenate(idxs, axis=1)
    ssum = jnp.sum(v, axis=1, keepdims=True) + 1e-6
    vals_ref[...] = v / ssum
    idx_ref[...] = i


def _body(xt_ref, w1_ref, b1_ref, w2_ref, b2_ref, vals_ref, idx_ref,
          r0, r1, r2, r3, sums_ref, sems):
    rings = (r0, r1, r2, r3)

    def start(b, h):
        pltpu.make_async_copy(
            xt_ref.at[b, pl.ds(h * _HBLK, _HBLK)],
            rings[b],
            sems.at[b],
        ).start()

    for b in range(_NBUF):
        start(b, 0)

    def step(h, _):
        for b in range(_NBUF):
            pltpu.make_async_copy(
                xt_ref.at[b, pl.ds(0, _HBLK)],
                rings[b],
                sems.at[b],
            ).wait()

            @pl.when(h + 1 < _NH)
            def _prefetch():
                start(b, h + 1)

            s = jnp.sum(rings[b][...], axis=0)
            part = jnp.sum(s, axis=0)

            @pl.when(h == 0)
            def _init():
                sums_ref[b, :] = part

            @pl.when(h != 0)
            def _acc():
                sums_ref[b, :] += part
        return _

    jax.lax.fori_loop(0, _NH, step, None)
    _route(sums_ref[...], w1_ref, b1_ref, w2_ref, b2_ref, vals_ref, idx_ref)


@jax.jit
def kernel(x, W1, gamma1, beta1, mean1, var1, W2, gamma2, beta2, mean2, var2):
    s1 = gamma1 * jax.lax.rsqrt(var1 + _EPS)
    s2 = gamma2 * jax.lax.rsqrt(var2 + _EPS)
    w1 = (W1 * s1[:, None]) * (1.0 / _HW)
    b1 = (beta1 - mean1 * s1)[None, :]
    w2 = W2 * s2[:, None]
    b2 = (beta2 - mean2 * s2)[None, :]

    xt = jnp.transpose(x, (0, 2, 3, 1))
    vals, idxs = pl.pallas_call(
        _body,
        in_specs=[
            pl.BlockSpec(memory_space=pl.ANY),
            pl.BlockSpec(memory_space=pltpu.VMEM),
            pl.BlockSpec(memory_space=pltpu.VMEM),
            pl.BlockSpec(memory_space=pltpu.VMEM),
            pl.BlockSpec(memory_space=pltpu.VMEM),
        ],
        out_specs=(
            pl.BlockSpec(memory_space=pltpu.VMEM),
            pl.BlockSpec(memory_space=pltpu.VMEM),
        ),
        out_shape=(
            jax.ShapeDtypeStruct((_B, _K), jnp.float32),
            jax.ShapeDtypeStruct((_B, _K), jnp.int32),
        ),
        scratch_shapes=[
            pltpu.VMEM((_HBLK, _W, _C), jnp.float32),
            pltpu.VMEM((_HBLK, _W, _C), jnp.float32),
            pltpu.VMEM((_HBLK, _W, _C), jnp.float32),
            pltpu.VMEM((_HBLK, _W, _C), jnp.float32),
            pltpu.VMEM((_B, _C), jnp.float32),
            pltpu.SemaphoreType.DMA((_NBUF,)),
        ],
    )(xt, w1, b1, w2, b2)
    return vals, idxs
